# Initial kernel scaffold; baseline (speedup 1.0000x reference)
import jax
import jax.numpy as jnp
from jax.experimental import pallas as pl


def kernel(y_pred):
    raise NotImplementedError("write your pallas kernel here")


# speedup vs baseline: 10.3990x; 10.3990x over previous
import functools

import jax
import jax.numpy as jnp
from jax.experimental import pallas as pl
from jax.experimental.pallas import tpu as pltpu
from jax.experimental.pallas import tpu_sc as plsc

N = 1000
C = 20
CPAD = 32
SLOT = 512
NMS_MAX = 400
TOP_K = 200
CONF = 0.01
IOU_T = 0.45
IMG = 512.0


def _decode_cols(g):
    cx = g(21) * g(29) * g(27) + g(25)
    cy = g(22) * g(30) * g(28) + g(26)
    w = jnp.exp(g(23) * g(31)) * g(27)
    h = jnp.exp(g(24) * g(32)) * g(28)
    xmin = (cx - 0.5 * w) * IMG
    ymin = (cy - 0.5 * h) * IMG
    xmax = (cx + 0.5 * w) * IMG
    ymax = (cy + 0.5 * h) * IMG
    return xmin, ymin, xmax, ymax


def _nms_tc_kernel(y_ref, slot_s_ref, slot_b_ref, m_ref, bt_ref, st_ref, k_ref):
    y = y_ref[0]
    yT = jnp.transpose(y)

    gC = lambda k: y[:, k:k + 1]
    gR = lambda k: yT[k:k + 1, :]
    xminC, yminC, xmaxC, ymaxC = _decode_cols(gC)
    xminR, yminR, xmaxR, ymaxR = _decode_cols(gR)

    x1 = jnp.maximum(xminC, xminR)
    y1 = jnp.maximum(yminC, yminR)
    x2 = jnp.minimum(xmaxC, xmaxR)
    y2 = jnp.minimum(ymaxC, ymaxR)
    inter = jnp.maximum(x2 - x1, 0.0) * jnp.maximum(y2 - y1, 0.0)
    aC = jnp.maximum(xmaxC - xminC, 0.0) * jnp.maximum(ymaxC - yminC, 0.0)
    aR = jnp.maximum(xmaxR - xminR, 0.0) * jnp.maximum(ymaxR - yminR, 0.0)
    union = aC + aR - inter
    safe = jnp.where(union > 0.0, union, 1.0)
    iou = jnp.where(union > 0.0, inter / safe, 0.0)
    m_ref[...] = (iou > IOU_T).astype(jnp.float32)

    ii = jax.lax.broadcasted_iota(jnp.int32, (N, N), 0)
    jj = jax.lax.broadcasted_iota(jnp.int32, (N, N), 1)
    boxesT = jnp.concatenate([xminR, yminR, xmaxR, ymaxR], axis=0)
    r_iota = jax.lax.broadcasted_iota(jnp.int32, (N, SLOT), 1).astype(jnp.float32)
    pad_row = jnp.where(
        jax.lax.broadcasted_iota(jnp.int32, (1, SLOT), 1) >= NMS_MAX, -1.0, 0.0)

    for c in range(C):
        sC = y[:, 1 + c:2 + c]
        sR = yT[1 + c:2 + c, :]
        validC = (sC > CONF).astype(jnp.float32)
        bt = ((sR > sC) | ((sR == sC) & (jj < ii))).astype(jnp.float32)
        bt_ref[...] = bt
        st_ref[...] = bt * m_ref[...]

        k_ref[...] = validC

        def body(carry):
            it, _ = carry
            K = k_ref[...]
            supp = jnp.dot(st_ref[...], K, preferred_element_type=jnp.float32)
            Kn = validC * (supp == 0.0).astype(jnp.float32)
            changed = jnp.sum(jnp.abs(Kn - K))
            k_ref[...] = Kn
            return (it + 1, (changed > 0.0).astype(jnp.int32))

        jax.lax.while_loop(lambda cr: cr[1] > 0, body, (0, jnp.int32(1)))

        K = k_ref[...]
        rank = jnp.dot(bt_ref[...], K, preferred_element_type=jnp.float32)
        vo = K * (rank < float(NMS_MAX)).astype(jnp.float32)
        E = vo * (rank == r_iota).astype(jnp.float32)
        slot_s = jnp.dot(sR, E, preferred_element_type=jnp.float32,
                         precision=jax.lax.Precision.HIGHEST)
        slot_b = jnp.dot(boxesT, E, preferred_element_type=jnp.float32,
                         precision=jax.lax.Precision.HIGHEST)
        slot_s_ref[0, c:c + 1, :] = slot_s + pad_row
        slot_b_ref[0, :, c, :] = slot_b

    slot_s_ref[0, C:CPAD, :] = jnp.full((CPAD - C, SLOT), -1.0, jnp.float32)
    slot_b_ref[0, :, C:CPAD, :] = jnp.zeros((4, CPAD - C, SLOT), jnp.float32)


def _phase_a(y_pred):
    B = y_pred.shape[0]
    return pl.pallas_call(
        _nms_tc_kernel,
        grid=(B,),
        in_specs=[pl.BlockSpec((1, N, 33), lambda b: (b, 0, 0))],
        out_specs=[
            pl.BlockSpec((1, CPAD, SLOT), lambda b: (b, 0, 0)),
            pl.BlockSpec((1, 4, CPAD, SLOT), lambda b: (b, 0, 0, 0)),
        ],
        out_shape=[
            jax.ShapeDtypeStruct((B, CPAD, SLOT), jnp.float32),
            jax.ShapeDtypeStruct((B, 4, CPAD, SLOT), jnp.float32),
        ],
        scratch_shapes=[
            pltpu.VMEM((N, N), jnp.float32),
            pltpu.VMEM((N, N), jnp.float32),
            pltpu.VMEM((N, N), jnp.float32),
            pltpu.VMEM((N, 1), jnp.float32),
        ],
    )(y_pred)


def _merge_sc_body(slot_s_hbm, slot_b_hbm, out_hbm, s_v, b_v, o_v):
    cid = jax.lax.axis_index("c")
    sid = jax.lax.axis_index("s")
    wid = sid * 2 + cid

    @pl.when(wid < 8)
    def _():
        pltpu.sync_copy(slot_s_hbm.at[wid], s_v)
        pltpu.sync_copy(slot_b_hbm.at[wid], b_v)
        lanes = jax.lax.iota(jnp.int32, 16)
        hi = lanes + 16
        d0 = jnp.clip(lanes - 2, 0, 3)
        big = jnp.int32(1 << 30)
        shift = SLOT.bit_length() - 1

        def step(t, carry):
            p0, p1 = carry
            h0 = plsc.load_gather(s_v, [lanes, p0])
            h1 = plsc.load_gather(s_v, [hi, p1])
            m = jnp.max(jnp.maximum(h0, h1))
            flat = jnp.minimum(
                jnp.min(jnp.where(h0 == m, lanes * SLOT + p0, big)),
                jnp.min(jnp.where(h1 == m, hi * SLOT + p1, big)))
            cls = jax.lax.shift_right_logical(flat, shift)
            rank = jax.lax.bitwise_and(flat, SLOT - 1)
            g = plsc.load_gather(
                b_v, [d0, jnp.full((16,), cls, jnp.int32),
                      jnp.full((16,), rank, jnp.int32)])
            valid = m > 0.0
            c_out = jnp.where(valid, cls.astype(jnp.float32) + 1.0, 1.0)
            v = jnp.where(lanes == 0, c_out, jnp.where(lanes == 1, m, g))
            plsc.store_scatter(o_v, [jnp.full((16,), t, jnp.int32), lanes],
                               v, mask=lanes < 6)
            pop0 = jnp.logical_and(lanes == cls, p0 == rank)
            pop1 = jnp.logical_and(hi == cls, p1 == rank)
            return (p0 + pop0.astype(jnp.int32), p1 + pop1.astype(jnp.int32))

        zeros = jnp.zeros((16,), jnp.int32)
        jax.lax.fori_loop(0, TOP_K, step, (zeros, zeros))
        pltpu.sync_copy(o_v, out_hbm.at[wid])


def _phase_b(slot_s, slot_b):
    B = slot_s.shape[0]
    mesh = plsc.VectorSubcoreMesh(core_axis_name="c", subcore_axis_name="s")
    fn = functools.partial(
        pl.kernel,
        mesh=mesh,
        compiler_params=pltpu.CompilerParams(needs_layout_passes=False),
        out_type=jax.ShapeDtypeStruct((B, TOP_K + 56, 8), jnp.float32),
        scratch_types=[
            pltpu.VMEM((CPAD, SLOT), jnp.float32),
            pltpu.VMEM((4, CPAD, SLOT), jnp.float32),
            pltpu.VMEM((TOP_K + 56, 8), jnp.float32),
        ],
    )(_merge_sc_body)
    return fn(slot_s, slot_b)


def kernel(y_pred):
    slot_s, slot_b = _phase_a(y_pred)
    out = _phase_b(slot_s, slot_b)
    return out[:, :TOP_K, :6]


# speedup vs baseline: 11.6692x; 1.1221x over previous
import functools

import jax
import jax.numpy as jnp
from jax.experimental import pallas as pl
from jax.experimental.pallas import tpu as pltpu
from jax.experimental.pallas import tpu_sc as plsc

N = 1000
C = 20
CPAD = 32
SLOT = 512
NMS_MAX = 400
TOP_K = 200
CONF = 0.01
IOU_T = 0.45
IMG = 512.0


def _decode_cols(g):
    cx = g(21) * g(29) * g(27) + g(25)
    cy = g(22) * g(30) * g(28) + g(26)
    w = jnp.exp(g(23) * g(31)) * g(27)
    h = jnp.exp(g(24) * g(32)) * g(28)
    xmin = (cx - 0.5 * w) * IMG
    ymin = (cy - 0.5 * h) * IMG
    xmax = (cx + 0.5 * w) * IMG
    ymax = (cy + 0.5 * h) * IMG
    return xmin, ymin, xmax, ymax


def _nms_tc_kernel(y_ref, slot_s_ref, slot_b_ref, m_ref, bt_ref, st_ref, k_ref):
    y = y_ref[0]
    yT = jnp.transpose(y)

    gC = lambda k: y[:, k:k + 1]
    gR = lambda k: yT[k:k + 1, :]
    xminC, yminC, xmaxC, ymaxC = _decode_cols(gC)
    xminR, yminR, xmaxR, ymaxR = _decode_cols(gR)

    x1 = jnp.maximum(xminC, xminR)
    y1 = jnp.maximum(yminC, yminR)
    x2 = jnp.minimum(xmaxC, xmaxR)
    y2 = jnp.minimum(ymaxC, ymaxR)
    inter = jnp.maximum(x2 - x1, 0.0) * jnp.maximum(y2 - y1, 0.0)
    aC = jnp.maximum(xmaxC - xminC, 0.0) * jnp.maximum(ymaxC - yminC, 0.0)
    aR = jnp.maximum(xmaxR - xminR, 0.0) * jnp.maximum(ymaxR - yminR, 0.0)
    union = aC + aR - inter
    safe = jnp.where(union > 0.0, union, 1.0)
    iou = jnp.where(union > 0.0, inter / safe, 0.0)
    m_ref[...] = (iou > IOU_T).astype(jnp.bfloat16)

    ii = jax.lax.broadcasted_iota(jnp.int32, (N, N), 0)
    jj = jax.lax.broadcasted_iota(jnp.int32, (N, N), 1)
    boxesT = jnp.concatenate([xminR, yminR, xmaxR, ymaxR], axis=0)
    r_iota = jax.lax.broadcasted_iota(jnp.int32, (N, SLOT), 1).astype(jnp.float32)
    pad_row = jnp.where(
        jax.lax.broadcasted_iota(jnp.int32, (1, SLOT), 1) >= NMS_MAX, -1.0, 0.0)

    for c in range(C):
        sC = y[:, 1 + c:2 + c]
        sR = yT[1 + c:2 + c, :]
        validC = (sC > CONF).astype(jnp.bfloat16)
        bt = ((sR > sC) | ((sR == sC) & (jj < ii))).astype(jnp.bfloat16)
        bt_ref[...] = bt
        st_ref[...] = bt * m_ref[...]

        k_ref[...] = validC

        def body(carry):
            it, _ = carry
            K = k_ref[...]
            supp = jnp.dot(st_ref[...], K, preferred_element_type=jnp.float32)
            Kn = validC * (supp == 0.0).astype(jnp.bfloat16)
            changed = jnp.sum(jnp.abs((Kn - K).astype(jnp.float32)))
            k_ref[...] = Kn
            return (it + 1, (changed > 0.0).astype(jnp.int32))

        jax.lax.while_loop(lambda cr: cr[1] > 0, body, (0, jnp.int32(1)))

        K = k_ref[...]
        rank = jnp.dot(bt_ref[...], K, preferred_element_type=jnp.float32)
        vo = K.astype(jnp.float32) * (rank < float(NMS_MAX)).astype(jnp.float32)
        E = vo * (rank == r_iota).astype(jnp.float32)
        cat5 = jnp.concatenate([sR, boxesT], axis=0)
        slot5 = jnp.dot(cat5, E, preferred_element_type=jnp.float32,
                        precision=jax.lax.Precision.HIGHEST)
        slot_s_ref[0, c:c + 1, :] = slot5[0:1] + pad_row
        slot_b_ref[0, :, c, :] = slot5[1:5]

    slot_s_ref[0, C:CPAD, :] = jnp.full((CPAD - C, SLOT), -1.0, jnp.float32)
    slot_b_ref[0, :, C:CPAD, :] = jnp.zeros((4, CPAD - C, SLOT), jnp.float32)


def _phase_a(y_pred):
    B = y_pred.shape[0]
    return pl.pallas_call(
        _nms_tc_kernel,
        grid=(B,),
        in_specs=[pl.BlockSpec((1, N, 33), lambda b: (b, 0, 0))],
        out_specs=[
            pl.BlockSpec((1, CPAD, SLOT), lambda b: (b, 0, 0)),
            pl.BlockSpec((1, 4, CPAD, SLOT), lambda b: (b, 0, 0, 0)),
        ],
        out_shape=[
            jax.ShapeDtypeStruct((B, CPAD, SLOT), jnp.float32),
            jax.ShapeDtypeStruct((B, 4, CPAD, SLOT), jnp.float32),
        ],
        scratch_shapes=[
            pltpu.VMEM((N, N), jnp.bfloat16),
            pltpu.VMEM((N, N), jnp.bfloat16),
            pltpu.VMEM((N, N), jnp.bfloat16),
            pltpu.VMEM((N, 1), jnp.bfloat16),
        ],
    )(y_pred)


def _merge_sc_body(slot_s_hbm, slot_b_hbm, out_hbm, s_v, b_v, o_v):
    cid = jax.lax.axis_index("c")
    sid = jax.lax.axis_index("s")
    wid = sid * 2 + cid

    @pl.when(wid < 8)
    def _():
        pltpu.sync_copy(slot_s_hbm.at[wid], s_v)
        pltpu.sync_copy(slot_b_hbm.at[wid], b_v)
        lanes = jax.lax.iota(jnp.int32, 16)
        hi = lanes + 16
        d0 = jnp.clip(lanes - 2, 0, 3)
        big = jnp.int32(1 << 30)
        shift = SLOT.bit_length() - 1

        def step(t, carry):
            p0, p1 = carry
            h0 = plsc.load_gather(s_v, [lanes, p0])
            h1 = plsc.load_gather(s_v, [hi, p1])
            m = jnp.max(jnp.maximum(h0, h1))
            flat = jnp.minimum(
                jnp.min(jnp.where(h0 == m, lanes * SLOT + p0, big)),
                jnp.min(jnp.where(h1 == m, hi * SLOT + p1, big)))
            cls = jax.lax.shift_right_logical(flat, shift)
            rank = jax.lax.bitwise_and(flat, SLOT - 1)
            g = plsc.load_gather(
                b_v, [d0, jnp.full((16,), cls, jnp.int32),
                      jnp.full((16,), rank, jnp.int32)])
            valid = m > 0.0
            c_out = jnp.where(valid, cls.astype(jnp.float32) + 1.0, 1.0)
            v = jnp.where(lanes == 0, c_out, jnp.where(lanes == 1, m, g))
            plsc.store_scatter(o_v, [jnp.full((16,), t, jnp.int32), lanes],
                               v, mask=lanes < 6)
            pop0 = jnp.logical_and(lanes == cls, p0 == rank)
            pop1 = jnp.logical_and(hi == cls, p1 == rank)
            return (p0 + pop0.astype(jnp.int32), p1 + pop1.astype(jnp.int32))

        zeros = jnp.zeros((16,), jnp.int32)
        jax.lax.fori_loop(0, TOP_K, step, (zeros, zeros))
        pltpu.sync_copy(o_v, out_hbm.at[wid])


def _phase_b(slot_s, slot_b):
    B = slot_s.shape[0]
    mesh = plsc.VectorSubcoreMesh(core_axis_name="c", subcore_axis_name="s")
    fn = functools.partial(
        pl.kernel,
        mesh=mesh,
        compiler_params=pltpu.CompilerParams(needs_layout_passes=False),
        out_type=jax.ShapeDtypeStruct((B, TOP_K + 56, 8), jnp.float32),
        scratch_types=[
            pltpu.VMEM((CPAD, SLOT), jnp.float32),
            pltpu.VMEM((4, CPAD, SLOT), jnp.float32),
            pltpu.VMEM((TOP_K + 56, 8), jnp.float32),
        ],
    )(_merge_sc_body)
    return fn(slot_s, slot_b)


def kernel(y_pred):
    slot_s, slot_b = _phase_a(y_pred)
    out = _phase_b(slot_s, slot_b)
    return out[:, :TOP_K, :6]


# speedup vs baseline: 13.0873x; 1.1215x over previous
import functools

import jax
import jax.numpy as jnp
from jax.experimental import pallas as pl
from jax.experimental.pallas import tpu as pltpu
from jax.experimental.pallas import tpu_sc as plsc

N = 1000
C = 20
CPAD = 32
SLOT = 512
NMS_MAX = 400
TOP_K = 200
CONF = 0.01
IOU_T = 0.45
IMG = 512.0


def _decode_cols(g):
    cx = g(21) * g(29) * g(27) + g(25)
    cy = g(22) * g(30) * g(28) + g(26)
    w = jnp.exp(g(23) * g(31)) * g(27)
    h = jnp.exp(g(24) * g(32)) * g(28)
    xmin = (cx - 0.5 * w) * IMG
    ymin = (cy - 0.5 * h) * IMG
    xmax = (cx + 0.5 * w) * IMG
    ymax = (cy + 0.5 * h) * IMG
    return xmin, ymin, xmax, ymax


def _nms_tc_kernel(y_ref, slot_s_ref, slot_b_ref, m_ref, bt_ref, st_ref, k_ref):
    y = y_ref[0]
    yT = jnp.transpose(y)

    gC = lambda k: y[:, k:k + 1]
    gR = lambda k: yT[k:k + 1, :]
    xminC, yminC, xmaxC, ymaxC = _decode_cols(gC)
    xminR, yminR, xmaxR, ymaxR = _decode_cols(gR)

    x1 = jnp.maximum(xminC, xminR)
    y1 = jnp.maximum(yminC, yminR)
    x2 = jnp.minimum(xmaxC, xmaxR)
    y2 = jnp.minimum(ymaxC, ymaxR)
    inter = jnp.maximum(x2 - x1, 0.0) * jnp.maximum(y2 - y1, 0.0)
    aC = jnp.maximum(xmaxC - xminC, 0.0) * jnp.maximum(ymaxC - yminC, 0.0)
    aR = jnp.maximum(xmaxR - xminR, 0.0) * jnp.maximum(ymaxR - yminR, 0.0)
    union = aC + aR - inter
    safe = jnp.where(union > 0.0, union, 1.0)
    iou = jnp.where(union > 0.0, inter / safe, 0.0)
    m_ref[...] = (iou > IOU_T).astype(jnp.bfloat16)

    ii = jax.lax.broadcasted_iota(jnp.int32, (N, N), 0)
    jj = jax.lax.broadcasted_iota(jnp.int32, (N, N), 1)
    boxesT = jnp.concatenate([xminR, yminR, xmaxR, ymaxR], axis=0)
    r_iota = jax.lax.broadcasted_iota(jnp.int32, (N, SLOT), 1).astype(jnp.float32)
    pad_row = jnp.where(
        jax.lax.broadcasted_iota(jnp.int32, (1, SLOT), 1) >= NMS_MAX, -1.0, 0.0)

    for c in range(C):
        sC = y[:, 1 + c:2 + c]
        sR = yT[1 + c:2 + c, :]
        validC = (sC > CONF).astype(jnp.bfloat16)
        bt = ((sR > sC) | ((sR == sC) & (jj < ii))).astype(jnp.bfloat16)
        bt_ref[...] = bt
        st = bt * m_ref[...]
        st_ref[...] = st
        validR = (sR > CONF).astype(jnp.bfloat16)
        supp1 = jnp.max(st * validR, axis=1, keepdims=True)
        k_ref[...] = validC * (supp1 == 0).astype(jnp.bfloat16)

        def body(carry):
            it, _ = carry
            K = k_ref[...]
            supp = jnp.dot(st_ref[...], K, preferred_element_type=jnp.float32)
            Kn = validC * (supp == 0.0).astype(jnp.bfloat16)
            changed = jnp.sum(jnp.abs((Kn - K).astype(jnp.float32)))
            k_ref[...] = Kn
            return (it + 1, (changed > 0.0).astype(jnp.int32))

        jax.lax.while_loop(lambda cr: cr[1] > 0, body, (0, jnp.int32(1)))

        K = k_ref[...]
        rank = jnp.dot(bt_ref[...], K, preferred_element_type=jnp.float32)
        vo = K.astype(jnp.float32) * (rank < float(NMS_MAX)).astype(jnp.float32)
        E = vo * (rank == r_iota).astype(jnp.float32)
        cat5 = jnp.concatenate([sR, boxesT], axis=0)
        slot5 = jnp.dot(cat5, E, preferred_element_type=jnp.float32,
                        precision=jax.lax.Precision.HIGHEST)
        slot_s_ref[0, c:c + 1, :] = slot5[0:1] + pad_row
        slot_b_ref[0, :, c, :] = slot5[1:5]

    slot_s_ref[0, C:CPAD, :] = jnp.full((CPAD - C, SLOT), -1.0, jnp.float32)
    slot_b_ref[0, :, C:CPAD, :] = jnp.zeros((4, CPAD - C, SLOT), jnp.float32)


def _phase_a(y_pred):
    B = y_pred.shape[0]
    return pl.pallas_call(
        _nms_tc_kernel,
        grid=(B,),
        in_specs=[pl.BlockSpec((1, N, 33), lambda b: (b, 0, 0))],
        out_specs=[
            pl.BlockSpec((1, CPAD, SLOT), lambda b: (b, 0, 0)),
            pl.BlockSpec((1, 4, CPAD, SLOT), lambda b: (b, 0, 0, 0)),
        ],
        out_shape=[
            jax.ShapeDtypeStruct((B, CPAD, SLOT), jnp.float32),
            jax.ShapeDtypeStruct((B, 4, CPAD, SLOT), jnp.float32),
        ],
        scratch_shapes=[
            pltpu.VMEM((N, N), jnp.bfloat16),
            pltpu.VMEM((N, N), jnp.bfloat16),
            pltpu.VMEM((N, N), jnp.bfloat16),
            pltpu.VMEM((N, 1), jnp.bfloat16),
        ],
    )(y_pred)


def _merge_sc_body(slot_s_hbm, slot_b_hbm, out_hbm, s_v, b_v, o_v):
    cid = jax.lax.axis_index("c")
    sid = jax.lax.axis_index("s")
    wid = sid * 2 + cid

    @pl.when(wid < 8)
    def _():
        pltpu.sync_copy(slot_s_hbm.at[wid], s_v)
        pltpu.sync_copy(slot_b_hbm.at[wid], b_v)
        lanes = jax.lax.iota(jnp.int32, 16)
        hi = lanes + 16
        d0 = jnp.clip(lanes - 2, 0, 3)
        big = jnp.int32(1 << 30)
        shift = SLOT.bit_length() - 1

        def step(t, carry):
            p0, p1 = carry
            h0 = plsc.load_gather(s_v, [lanes, p0])
            h1 = plsc.load_gather(s_v, [hi, p1])
            m = jnp.max(jnp.maximum(h0, h1))
            flat = jnp.minimum(
                jnp.min(jnp.where(h0 == m, lanes * SLOT + p0, big)),
                jnp.min(jnp.where(h1 == m, hi * SLOT + p1, big)))
            cls = jax.lax.shift_right_logical(flat, shift)
            rank = jax.lax.bitwise_and(flat, SLOT - 1)
            g = plsc.load_gather(
                b_v, [d0, jnp.full((16,), cls, jnp.int32),
                      jnp.full((16,), rank, jnp.int32)])
            valid = m > 0.0
            c_out = jnp.where(valid, cls.astype(jnp.float32) + 1.0, 1.0)
            v = jnp.where(lanes == 0, c_out, jnp.where(lanes == 1, m, g))
            plsc.store_scatter(o_v, [jnp.full((16,), t, jnp.int32), lanes],
                               v, mask=lanes < 6)
            pop0 = jnp.logical_and(lanes == cls, p0 == rank)
            pop1 = jnp.logical_and(hi == cls, p1 == rank)
            return (p0 + pop0.astype(jnp.int32), p1 + pop1.astype(jnp.int32))

        zeros = jnp.zeros((16,), jnp.int32)
        jax.lax.fori_loop(0, TOP_K, step, (zeros, zeros))
        pltpu.sync_copy(o_v, out_hbm.at[wid])


def _phase_b(slot_s, slot_b):
    B = slot_s.shape[0]
    mesh = plsc.VectorSubcoreMesh(core_axis_name="c", subcore_axis_name="s")
    fn = functools.partial(
        pl.kernel,
        mesh=mesh,
        compiler_params=pltpu.CompilerParams(needs_layout_passes=False),
        out_type=jax.ShapeDtypeStruct((B, TOP_K + 56, 8), jnp.float32),
        scratch_types=[
            pltpu.VMEM((CPAD, SLOT), jnp.float32),
            pltpu.VMEM((4, CPAD, SLOT), jnp.float32),
            pltpu.VMEM((TOP_K + 56, 8), jnp.float32),
        ],
    )(_merge_sc_body)
    return fn(slot_s, slot_b)


def kernel(y_pred):
    slot_s, slot_b = _phase_a(y_pred)
    out = _phase_b(slot_s, slot_b)
    return out[:, :TOP_K, :6]


# speedup vs baseline: 18.4746x; 1.4116x over previous
import functools

import jax
import jax.numpy as jnp
from jax.experimental import pallas as pl
from jax.experimental.pallas import tpu as pltpu
from jax.experimental.pallas import tpu_sc as plsc

N = 1000
NPAD = 1024
C = 20
CPAD = 32
SLOT = 512
NMS_MAX = 400
TOP_K = 200
CONF = 0.01
IOU_T = 0.45
IMG = 512.0
EMPTY = SLOT - 1


def _decode_cols(g):
    cx = g(21) * g(29) * g(27) + g(25)
    cy = g(22) * g(30) * g(28) + g(26)
    w = jnp.exp(g(23) * g(31)) * g(27)
    h = jnp.exp(g(24) * g(32)) * g(28)
    xmin = (cx - 0.5 * w) * IMG
    ymin = (cy - 0.5 * h) * IMG
    xmax = (cx + 0.5 * w) * IMG
    ymax = (cy + 0.5 * h) * IMG
    return xmin, ymin, xmax, ymax


def _nms_tc_kernel(y_ref, scores_ref, boxes_ref, rk_ref,
                   m_ref, bt_ref, st_ref, k_ref):
    y = y_ref[0]
    yT = jnp.transpose(y)

    gC = lambda k: y[:, k:k + 1]
    gR = lambda k: yT[k:k + 1, :]
    xminC, yminC, xmaxC, ymaxC = _decode_cols(gC)
    xminR, yminR, xmaxR, ymaxR = _decode_cols(gR)

    x1 = jnp.maximum(xminC, xminR)
    y1 = jnp.maximum(yminC, yminR)
    x2 = jnp.minimum(xmaxC, xmaxR)
    y2 = jnp.minimum(ymaxC, ymaxR)
    inter = jnp.maximum(x2 - x1, 0.0) * jnp.maximum(y2 - y1, 0.0)
    aC = jnp.maximum(xmaxC - xminC, 0.0) * jnp.maximum(ymaxC - yminC, 0.0)
    aR = jnp.maximum(xmaxR - xminR, 0.0) * jnp.maximum(ymaxR - yminR, 0.0)
    union = aC + aR - inter
    safe = jnp.where(union > 0.0, union, 1.0)
    iou = jnp.where(union > 0.0, inter / safe, 0.0)
    m_ref[...] = (iou > IOU_T).astype(jnp.bfloat16)

    ii = jax.lax.broadcasted_iota(jnp.int32, (N, N), 0)
    jj = jax.lax.broadcasted_iota(jnp.int32, (N, N), 1)

    scores_ref[0, :, :N] = yT[1:1 + C, :]
    scores_ref[0, :, N:] = jnp.zeros((C, NPAD - N), jnp.float32)
    boxes_ref[0, :, :N] = jnp.concatenate([xminR, yminR, xmaxR, ymaxR], axis=0)
    boxes_ref[0, :, N:] = jnp.zeros((4, NPAD - N), jnp.float32)
    rk_ref[0, N:, :] = jnp.full((NPAD - N, CPAD), EMPTY, jnp.int32)

    for c in range(C):
        sC = y[:, 1 + c:2 + c]
        sR = yT[1 + c:2 + c, :]
        validC = (sC > CONF).astype(jnp.bfloat16)
        bt = ((sR > sC) | ((sR == sC) & (jj < ii))).astype(jnp.bfloat16)
        bt_ref[...] = bt
        st = bt * m_ref[...]
        st_ref[...] = st
        validR = (sR > CONF).astype(jnp.bfloat16)
        supp1 = jnp.max(st * validR, axis=1, keepdims=True)
        k_ref[...] = validC * (supp1 == 0).astype(jnp.bfloat16)

        def body(carry):
            it, _ = carry
            K = k_ref[...]
            supp = jnp.dot(st_ref[...], K, preferred_element_type=jnp.float32)
            Kn = validC * (supp == 0.0).astype(jnp.bfloat16)
            changed = jnp.sum(jnp.abs((Kn - K).astype(jnp.float32)))
            k_ref[...] = Kn
            return (it + 1, (changed > 0.0).astype(jnp.int32))

        jax.lax.while_loop(lambda cr: cr[1] > 0, body, (0, jnp.int32(1)))

        K = k_ref[...]
        rank = jnp.dot(bt_ref[...], K, preferred_element_type=jnp.float32)
        kept = (K.astype(jnp.float32) > 0.0) & (rank < float(NMS_MAX))
        rk_ref[0, :N, c:c + 1] = jnp.where(kept, rank, float(EMPTY)).astype(
            jnp.int32)


def _phase_a(y_pred):
    B = y_pred.shape[0]
    return pl.pallas_call(
        _nms_tc_kernel,
        grid=(B,),
        in_specs=[pl.BlockSpec((1, N, 33), lambda b: (b, 0, 0))],
        out_specs=[
            pl.BlockSpec((1, C, NPAD), lambda b: (b, 0, 0)),
            pl.BlockSpec((1, 4, NPAD), lambda b: (b, 0, 0)),
            pl.BlockSpec((1, NPAD, CPAD), lambda b: (b, 0, 0)),
        ],
        out_shape=[
            jax.ShapeDtypeStruct((B, C, NPAD), jnp.float32),
            jax.ShapeDtypeStruct((B, 4, NPAD), jnp.float32),
            jax.ShapeDtypeStruct((B, NPAD, CPAD), jnp.int32),
        ],
        scratch_shapes=[
            pltpu.VMEM((N, N), jnp.bfloat16),
            pltpu.VMEM((N, N), jnp.bfloat16),
            pltpu.VMEM((N, N), jnp.bfloat16),
            pltpu.VMEM((N, 1), jnp.bfloat16),
        ],
    )(y_pred)


def _merge_sc_body(scores_hbm, boxes_hbm, rk_hbm, out_hbm,
                   sc_v, bx_v, rk_v, slots_v, sidx_v, o_v):
    cid = jax.lax.axis_index("c")
    sid = jax.lax.axis_index("s")
    wid = sid * 2 + cid

    @pl.when(wid < 8)
    def _():
        pltpu.sync_copy(scores_hbm.at[wid], sc_v)
        pltpu.sync_copy(boxes_hbm.at[wid], bx_v)
        pltpu.sync_copy(rk_hbm.at[wid], rk_v)
        lanes = jax.lax.iota(jnp.int32, 16)
        d0 = jnp.clip(lanes - 2, 0, 3)
        big = jnp.int32(1 << 30)

        def init_step(i, _):
            flat = i * 16 + lanes
            val = jnp.where((jax.lax.bitwise_and(flat, SLOT - 1) < NMS_MAX)
                            & (flat < C * SLOT), 0.0, -1.0)
            plsc.store_scatter(slots_v, [flat], val)
            return 0

        jax.lax.fori_loop(0, CPAD * SLOT // 16, init_step, 0)

        def scat_step(ck, _):
            c = ck // 64
            k = ck - c * 64
            col = k * 16 + lanes
            rkv = plsc.load_gather(rk_v, [col * CPAD + c])
            sv = plsc.load_gather(sc_v, [c * NPAD + col])
            dst = c * SLOT + rkv
            plsc.store_scatter(slots_v, [dst], sv)
            plsc.store_scatter(sidx_v, [dst], col)
            return 0

        jax.lax.fori_loop(0, C * (NPAD // 16), scat_step, 0)

        def step(t, carry):
            p0, p1 = carry
            h0 = plsc.load_gather(slots_v, [lanes * SLOT + p0])
            h1 = plsc.load_gather(slots_v, [(lanes + 16) * SLOT + p1])
            m = jnp.max(jnp.maximum(h0, h1))
            flat = jnp.minimum(
                jnp.min(jnp.where(h0 == m, lanes * SLOT + p0, big)),
                jnp.min(jnp.where(h1 == m, (lanes + 16) * SLOT + p1, big)))
            cls = jax.lax.shift_right_logical(flat, SLOT.bit_length() - 1)
            bidv = plsc.load_gather(sidx_v, [jnp.full((16,), flat, jnp.int32)])
            bid = jnp.clip(jnp.max(bidv), 0, NPAD - 1)
            g = plsc.load_gather(bx_v, [d0 * NPAD + bid])
            validv = jnp.full((16,), m, jnp.float32) > 0.0
            c_out = jnp.where(m > 0.0, cls.astype(jnp.float32) + 1.0, 1.0)
            v = jnp.where(lanes == 0, c_out,
                          jnp.where(lanes == 1, m,
                                    jnp.where(validv, g, 0.0)))
            plsc.store_scatter(o_v, [t * 8 + lanes], v, mask=lanes < 6)
            pop = flat - cls * SLOT
            pop0 = (lanes == cls) & (p0 == pop)
            pop1 = ((lanes + 16) == cls) & (p1 == pop)
            return (p0 + pop0.astype(jnp.int32), p1 + pop1.astype(jnp.int32))

        zeros = jnp.zeros((16,), jnp.int32)
        jax.lax.fori_loop(0, TOP_K, step, (zeros, zeros))
        pltpu.sync_copy(o_v, out_hbm.at[wid])


def _phase_b(scores, boxes, rk):
    B = scores.shape[0]
    mesh = plsc.VectorSubcoreMesh(core_axis_name="c", subcore_axis_name="s")
    fn = functools.partial(
        pl.kernel,
        mesh=mesh,
        compiler_params=pltpu.CompilerParams(needs_layout_passes=False),
        out_type=jax.ShapeDtypeStruct((B, (TOP_K + 56) * 8), jnp.float32),
        scratch_types=[
            pltpu.VMEM((C * NPAD,), jnp.float32),
            pltpu.VMEM((4 * NPAD,), jnp.float32),
            pltpu.VMEM((NPAD * CPAD,), jnp.int32),
            pltpu.VMEM((CPAD * SLOT,), jnp.float32),
            pltpu.VMEM((CPAD * SLOT,), jnp.int32),
            pltpu.VMEM(((TOP_K + 56) * 8,), jnp.float32),
        ],
    )(_merge_sc_body)
    return fn(scores.reshape(B, C * NPAD), boxes.reshape(B, 4 * NPAD),
              rk.reshape(B, NPAD * CPAD))


def kernel(y_pred):
    scores, boxes, rk = _phase_a(y_pred)
    out = _phase_b(scores, boxes, rk)
    return out.reshape(-1, TOP_K + 56, 8)[:, :TOP_K, :6]
